# baseline (speedup 1.0000x reference)
import functools

import jax
import jax.numpy as jnp
from jax import lax
from jax.experimental import pallas as pl
from jax.experimental.pallas import tpu as pltpu
from jax.experimental.pallas import tpu_sc as plsc

_VOCAB = 100000
_S = 200
_D = 64
_B = 4096

_NC = 2
_NS = 16
_NW = _NC * _NS

_BW = _B // _NW
_NBUF = 4

_mesh = plsc.VectorSubcoreMesh(core_axis_name="c", subcore_axis_name="s")


@functools.partial(
    pl.kernel,
    mesh=_mesh,
    out_type=jax.ShapeDtypeStruct((_S, _D // 8, _NW, 8, _BW), jnp.float32),
    scratch_types=[
        pltpu.VMEM((_S, _D), jnp.float32),
        pltpu.VMEM((_S, _BW), jnp.int32),
    ]
    + [pltpu.VMEM((_BW, _D), jnp.float32) for _ in range(_NBUF)]
    + [pltpu.VMEM((8, 8, _BW), jnp.float32) for _ in range(_NBUF)]
    + [pltpu.SemaphoreType.DMA for _ in range(2 * _NBUF)],
    compiler_params=pltpu.CompilerParams(use_tc_tiling_on_sc=False,
                                         needs_layout_passes=False),
)
def _embed_pe(xt_hbm, emb_hbm, pos_hbm, out_hbm, pos_v, idx_v,
              gb0, gb1, gb2, gb3, sb0, sb1, sb2, sb3,
              g0, g1, g2, g3, s0, s1, s2, s3):
    gbufs = [gb0, gb1, gb2, gb3]
    sbufs = [sb0, sb1, sb2, sb3]
    gsems = [g0, g1, g2, g3]
    ssems = [s0, s1, s2, s3]

    cid = lax.axis_index("c")
    sid = lax.axis_index("s")
    wid = sid * _NC + cid

    pltpu.sync_copy(pos_hbm, pos_v)
    pltpu.sync_copy(xt_hbm.at[:, wid], idx_v)

    rowc = [lax.iota(jnp.int32, 16) + 16 * blk for blk in range(_BW // 16)]

    def issue_gather(s, p):
        pltpu.async_copy(emb_hbm.at[idx_v.at[s]], gbufs[p], gsems[p])

    def wait_gather(p):
        pltpu.make_async_copy(emb_hbm.at[pl.ds(0, _BW)], gbufs[p],
                              gsems[p]).wait()

    def issue_scatter(s, p):
        pltpu.async_copy(sbufs[p], out_hbm.at[s, :, wid], ssems[p])

    def wait_scatter(p):
        pltpu.make_async_copy(sbufs[p], out_hbm.at[0, :, 0], ssems[p]).wait()

    def transpose_add(s, p):
        gb = gbufs[p]
        sb = sbufs[p]
        svec = jnp.full((16,), s, jnp.int32)

        for j in range(_D // 16):
            def dd_body(dd, carry, j=j):
                d = j * 16 + dd
                dvec = jnp.full((16,), d, jnp.int32)
                ps = plsc.load_gather(pos_v, [svec, dvec])
                for blk in range(_BW // 16):
                    v = plsc.load_gather(gb, [rowc[blk], dvec])
                    sb[d >> 3, d & 7, pl.ds(16 * blk, 16)] = v + ps
                return carry

            lax.fori_loop(0, 16, dd_body, 0)

    def body(s, p):
        @pl.when(s + (_NBUF - 1) < _S)
        def _():
            issue_gather(s + (_NBUF - 1), (p + _NBUF - 1) % _NBUF)

        wait_gather(p)

        @pl.when(s >= _NBUF)
        def _():
            wait_scatter(p)

        transpose_add(s, p)
        issue_scatter(s, p)

    for p in range(_NBUF - 1):
        issue_gather(p, p)

    def outer(i, carry):
        for p in range(_NBUF):
            body(i * _NBUF + p, p)
        return carry

    lax.fori_loop(0, _S // _NBUF, outer, 0)
    for p in range(_NBUF):
        wait_scatter(p)


def kernel(x, emb_table, pos_table):
    xt = jnp.transpose(x.astype(jnp.int32)).reshape(_S, _NW, _BW)
    out5 = _embed_pe(xt, emb_table, pos_table)
    return jnp.transpose(out5, (2, 4, 0, 1, 3)).reshape(_B, _S, _D)


# speedup vs baseline: 1.8893x; 1.8893x over previous
import functools

import jax
import jax.numpy as jnp
from jax import lax
from jax.experimental import pallas as pl
from jax.experimental.pallas import tpu as pltpu
from jax.experimental.pallas import tpu_sc as plsc

_VOCAB = 100000
_S = 200
_D = 64
_B = 4096

_NC = 2
_NS = 16
_NW = _NC * _NS

_BW = _B // _NW
_NBUF = 4

_mesh = plsc.VectorSubcoreMesh(core_axis_name="c", subcore_axis_name="s")


@functools.partial(
    pl.kernel,
    mesh=_mesh,
    out_type=jax.ShapeDtypeStruct((_S, _D // 8, _NW, 8, _BW), jnp.float32),
    scratch_types=[
        pltpu.VMEM((_S, _D), jnp.float32),
        pltpu.VMEM((_S, _BW), jnp.int32),
    ]
    + [pltpu.VMEM((_BW, _D), jnp.float32) for _ in range(_NBUF)]
    + [pltpu.VMEM((8, 8, _BW), jnp.float32) for _ in range(_NBUF)]
    + [pltpu.SemaphoreType.DMA for _ in range(2 * _NBUF)],
    compiler_params=pltpu.CompilerParams(use_tc_tiling_on_sc=False,
                                         needs_layout_passes=False),
)
def _embed_pe(xt_hbm, emb_hbm, pos_hbm, out_hbm, pos_v, idx_v,
              gb0, gb1, gb2, gb3, sb0, sb1, sb2, sb3,
              g0, g1, g2, g3, s0, s1, s2, s3):
    gbufs = [gb0, gb1, gb2, gb3]
    sbufs = [sb0, sb1, sb2, sb3]
    gsems = [g0, g1, g2, g3]
    ssems = [s0, s1, s2, s3]

    cid = lax.axis_index("c")
    sid = lax.axis_index("s")
    wid = sid * _NC + cid

    pltpu.sync_copy(pos_hbm, pos_v)
    pltpu.sync_copy(xt_hbm.at[:, wid], idx_v)

    rowc = [lax.iota(jnp.int32, 16) + 16 * blk for blk in range(_BW // 16)]

    def issue_gather(s, p):
        pltpu.async_copy(emb_hbm.at[idx_v.at[s]], gbufs[p], gsems[p])

    def wait_gather(p):
        pltpu.make_async_copy(emb_hbm.at[pl.ds(0, _BW)], gbufs[p],
                              gsems[p]).wait()

    def issue_scatter(s, p):
        pltpu.async_copy(sbufs[p], out_hbm.at[s, :, wid], ssems[p])

    def wait_scatter(p):
        pltpu.make_async_copy(sbufs[p], out_hbm.at[0, :, 0], ssems[p]).wait()

    def transpose_add(s, p):
        gb = gbufs[p]
        sb = sbufs[p]
        svec = jnp.full((16,), s, jnp.int32)

        @plsc.parallel_loop(0, _D, 1, unroll=4)
        def _d_body(d):
            dvec = jnp.full((16,), d, jnp.int32)
            ps = plsc.load_gather(pos_v, [svec, dvec])
            for blk in range(_BW // 16):
                v = plsc.load_gather(gb, [rowc[blk], dvec])
                sb[d >> 3, d & 7, pl.ds(16 * blk, 16)] = v + ps

    def body(s, p):
        @pl.when(s + (_NBUF - 1) < _S)
        def _():
            issue_gather(s + (_NBUF - 1), (p + _NBUF - 1) % _NBUF)

        wait_gather(p)

        @pl.when(s >= _NBUF)
        def _():
            wait_scatter(p)

        transpose_add(s, p)
        issue_scatter(s, p)

    for p in range(_NBUF - 1):
        issue_gather(p, p)

    def outer(i, carry):
        for p in range(_NBUF):
            body(i * _NBUF + p, p)
        return carry

    lax.fori_loop(0, _S // _NBUF, outer, 0)
    for p in range(_NBUF):
        wait_scatter(p)


def kernel(x, emb_table, pos_table):
    xt = jnp.transpose(x.astype(jnp.int32)).reshape(_S, _NW, _BW)
    out5 = _embed_pe(xt, emb_table, pos_table)
    return jnp.transpose(out5, (2, 4, 0, 1, 3)).reshape(_B, _S, _D)


# speedup vs baseline: 6.3608x; 3.3668x over previous
import functools

import jax
import jax.numpy as jnp
from jax import lax
from jax.experimental import pallas as pl
from jax.experimental.pallas import tpu as pltpu
from jax.experimental.pallas import tpu_sc as plsc

_VOCAB = 100000
_S = 200
_D = 64
_B = 4096

_NC = 2
_NS = 16
_NW = _NC * _NS

_BW = _B // _NW
_NBUF = 4

_mesh = plsc.VectorSubcoreMesh(core_axis_name="c", subcore_axis_name="s")


@functools.partial(
    pl.kernel,
    mesh=_mesh,
    out_type=jax.ShapeDtypeStruct((_S, _D // 8, _NW, 8, _BW), jnp.float32),
    scratch_types=[
        pltpu.VMEM((_S, _D), jnp.float32),
        pltpu.VMEM((_S, _BW), jnp.int32),
    ]
    + [pltpu.VMEM((_BW, _D), jnp.float32) for _ in range(_NBUF)]
    + [pltpu.VMEM((8, 8, _BW), jnp.float32) for _ in range(_NBUF)]
    + [pltpu.VMEM((_D * (_BW + 1),), jnp.float32)]
    + [pltpu.SemaphoreType.DMA for _ in range(2 * _NBUF)],
    compiler_params=pltpu.CompilerParams(use_tc_tiling_on_sc=False,
                                         needs_layout_passes=False),
)
def _embed_pe(xt_hbm, emb_hbm, pos_hbm, out_hbm, pos_v, idx_v,
              gb0, gb1, gb2, gb3, sb0, sb1, sb2, sb3, wb,
              g0, g1, g2, g3, s0, s1, s2, s3):
    gbufs = [gb0, gb1, gb2, gb3]
    sbufs = [sb0, sb1, sb2, sb3]
    gsems = [g0, g1, g2, g3]
    ssems = [s0, s1, s2, s3]

    cid = lax.axis_index("c")
    sid = lax.axis_index("s")
    wid = sid * _NC + cid

    pltpu.sync_copy(pos_hbm, pos_v)
    pltpu.sync_copy(xt_hbm.at[:, wid], idx_v)

    rowc = [lax.iota(jnp.int32, 16) + 16 * blk for blk in range(_BW // 16)]

    def issue_gather(s, p):
        pltpu.async_copy(emb_hbm.at[idx_v.at[s]], gbufs[p], gsems[p])

    def wait_gather(p):
        pltpu.make_async_copy(emb_hbm.at[pl.ds(0, _BW)], gbufs[p],
                              gsems[p]).wait()

    def issue_scatter(s, p):
        pltpu.async_copy(sbufs[p], out_hbm.at[s, :, wid], ssems[p])

    def wait_scatter(p):
        pltpu.make_async_copy(sbufs[p], out_hbm.at[0, :, 0], ssems[p]).wait()

    basej = [(16 * j + lax.iota(jnp.int32, 16)) * (_BW + 1)
             for j in range(_D // 16)]

    def transpose_add(s, p):
        gb = gbufs[p]
        sb = sbufs[p]
        pv = [pos_v[s, pl.ds(16 * j, 16)] for j in range(_D // 16)]

        @plsc.parallel_loop(0, _BW, 1, unroll=2)
        def _r_body(r):
            rv = jnp.full((16,), r, jnp.int32)
            for j in range(_D // 16):
                v = gb[r, pl.ds(16 * j, 16)]
                plsc.store_scatter(wb, [basej[j] + rv], v + pv[j])

        @plsc.parallel_loop(0, _D, 1, unroll=2)
        def _d_body(d):
            for k in range(_BW // 16):
                sb[d >> 3, d & 7, pl.ds(16 * k, 16)] = (
                    wb[pl.ds(d * (_BW + 1) + 16 * k, 16)])

    def body(s, p):
        @pl.when(s + (_NBUF - 1) < _S)
        def _():
            issue_gather(s + (_NBUF - 1), (p + _NBUF - 1) % _NBUF)

        wait_gather(p)

        @pl.when(s >= _NBUF)
        def _():
            wait_scatter(p)

        transpose_add(s, p)
        issue_scatter(s, p)

    for p in range(_NBUF - 1):
        issue_gather(p, p)

    def outer(i, carry):
        for p in range(_NBUF):
            body(i * _NBUF + p, p)
        return carry

    lax.fori_loop(0, _S // _NBUF, outer, 0)
    for p in range(_NBUF):
        wait_scatter(p)


def kernel(x, emb_table, pos_table):
    xt = jnp.transpose(x.astype(jnp.int32)).reshape(_S, _NW, _BW)
    out5 = _embed_pe(xt, emb_table, pos_table)
    return jnp.transpose(out5, (2, 4, 0, 1, 3)).reshape(_B, _S, _D)


# speedup vs baseline: 6.3808x; 1.0031x over previous
import functools

import jax
import jax.numpy as jnp
from jax import lax
from jax.experimental import pallas as pl
from jax.experimental.pallas import tpu as pltpu
from jax.experimental.pallas import tpu_sc as plsc

_VOCAB = 100000
_S = 200
_D = 64
_B = 4096

_NC = 2
_NS = 16
_NW = _NC * _NS

_BW = _B // _NW
_NBUF = 4

_mesh = plsc.VectorSubcoreMesh(core_axis_name="c", subcore_axis_name="s")


@functools.partial(
    pl.kernel,
    mesh=_mesh,
    out_type=jax.ShapeDtypeStruct((_S, _D // 8, _NW, 8, _BW), jnp.float32),
    scratch_types=[
        pltpu.VMEM((_S, _D), jnp.float32),
        pltpu.VMEM((_S, _BW), jnp.int32),
    ]
    + [pltpu.VMEM((_BW, _D), jnp.float32) for _ in range(_NBUF)]
    + [pltpu.VMEM((8, 8, _BW), jnp.float32) for _ in range(_NBUF)]
    + [pltpu.VMEM((_D * (_BW + 1),), jnp.float32)]
    + [pltpu.SemaphoreType.DMA for _ in range(2 * _NBUF)],
    compiler_params=pltpu.CompilerParams(use_tc_tiling_on_sc=False,
                                         needs_layout_passes=False),
)
def _embed_pe(xt_hbm, emb_hbm, pos_hbm, out_hbm, pos_v, idx_v,
              gb0, gb1, gb2, gb3, sb0, sb1, sb2, sb3, wb,
              g0, g1, g2, g3, s0, s1, s2, s3):
    gbufs = [gb0, gb1, gb2, gb3]
    sbufs = [sb0, sb1, sb2, sb3]
    gsems = [g0, g1, g2, g3]
    ssems = [s0, s1, s2, s3]

    cid = lax.axis_index("c")
    sid = lax.axis_index("s")
    wid = sid * _NC + cid

    pltpu.sync_copy(pos_hbm, pos_v)
    pltpu.sync_copy(xt_hbm.at[:, wid], idx_v)

    rowc = [lax.iota(jnp.int32, 16) + 16 * blk for blk in range(_BW // 16)]

    def issue_gather(s, p):
        pltpu.async_copy(emb_hbm.at[idx_v.at[s]], gbufs[p], gsems[p])

    def wait_gather(p):
        pltpu.make_async_copy(emb_hbm.at[pl.ds(0, _BW)], gbufs[p],
                              gsems[p]).wait()

    def issue_scatter(s, p):
        pltpu.async_copy(sbufs[p], out_hbm.at[s, :, wid], ssems[p])

    def wait_scatter(p):
        pltpu.make_async_copy(sbufs[p], out_hbm.at[0, :, 0], ssems[p]).wait()

    basej = [(16 * j + lax.iota(jnp.int32, 16)) * (_BW + 1)
             for j in range(_D // 16)]

    def transpose_add(s, p):
        gb = gbufs[p]
        sb = sbufs[p]
        pv = [pos_v[s, pl.ds(16 * j, 16)] for j in range(_D // 16)]

        @plsc.parallel_loop(0, _BW, 1, unroll=4)
        def _r_body(r):
            rv = jnp.full((16,), r, jnp.int32)
            for j in range(_D // 16):
                v = gb[r, pl.ds(16 * j, 16)]
                plsc.store_scatter(wb, [basej[j] + rv], v + pv[j])

        @plsc.parallel_loop(0, _D, 1, unroll=4)
        def _d_body(d):
            for k in range(_BW // 16):
                sb[d >> 3, d & 7, pl.ds(16 * k, 16)] = (
                    wb[pl.ds(d * (_BW + 1) + 16 * k, 16)])

    def body(s, p):
        @pl.when(s + (_NBUF - 1) < _S)
        def _():
            issue_gather(s + (_NBUF - 1), (p + _NBUF - 1) % _NBUF)

        wait_gather(p)

        @pl.when(s >= _NBUF)
        def _():
            wait_scatter(p)

        transpose_add(s, p)
        issue_scatter(s, p)

    for p in range(_NBUF - 1):
        issue_gather(p, p)

    def outer(i, carry):
        for p in range(_NBUF):
            body(i * _NBUF + p, p)
        return carry

    lax.fori_loop(0, _S // _NBUF, outer, 0)
    for p in range(_NBUF):
        wait_scatter(p)


def kernel(x, emb_table, pos_table):
    xt = jnp.transpose(x.astype(jnp.int32)).reshape(_S, _NW, _BW)
    out5 = _embed_pe(xt, emb_table, pos_table)
    return jnp.transpose(out5, (2, 4, 0, 1, 3)).reshape(_B, _S, _D)


# speedup vs baseline: 6.3889x; 1.0013x over previous
import functools

import jax
import jax.numpy as jnp
from jax import lax
from jax.experimental import pallas as pl
from jax.experimental.pallas import tpu as pltpu
from jax.experimental.pallas import tpu_sc as plsc

_VOCAB = 100000
_S = 200
_D = 64
_B = 4096

_NC = 2
_NS = 16
_NW = _NC * _NS

_BW = _B // _NW
_NBUF = 4

_mesh = plsc.VectorSubcoreMesh(core_axis_name="c", subcore_axis_name="s")


@functools.partial(
    pl.kernel,
    mesh=_mesh,
    out_type=jax.ShapeDtypeStruct((_S, _D // 8, _NW, 8, _BW), jnp.float32),
    scratch_types=[
        pltpu.VMEM((_S, _D), jnp.float32),
        pltpu.VMEM((_S, _BW), jnp.int32),
    ]
    + [pltpu.VMEM((_BW, _D), jnp.float32) for _ in range(_NBUF)]
    + [pltpu.VMEM((8, 8, _BW), jnp.float32) for _ in range(_NBUF)]
    + [pltpu.VMEM((_D * (_BW + 1),), jnp.float32)]
    + [pltpu.SemaphoreType.DMA for _ in range(2 * _NBUF)],
    compiler_params=pltpu.CompilerParams(use_tc_tiling_on_sc=False,
                                         needs_layout_passes=False),
)
def _embed_pe(xt_hbm, emb_hbm, pos_hbm, out_hbm, pos_v, idx_v,
              gb0, gb1, gb2, gb3, sb0, sb1, sb2, sb3, wb,
              g0, g1, g2, g3, s0, s1, s2, s3):
    gbufs = [gb0, gb1, gb2, gb3]
    sbufs = [sb0, sb1, sb2, sb3]
    gsems = [g0, g1, g2, g3]
    ssems = [s0, s1, s2, s3]

    cid = lax.axis_index("c")
    sid = lax.axis_index("s")
    wid = sid * _NC + cid

    pltpu.sync_copy(pos_hbm, pos_v)
    pltpu.sync_copy(xt_hbm.at[:, wid], idx_v)

    def issue_gather(s, p):
        pltpu.async_copy(emb_hbm.at[idx_v.at[s]], gbufs[p], gsems[p])

    def wait_gather(p):
        pltpu.make_async_copy(emb_hbm.at[pl.ds(0, _BW)], gbufs[p],
                              gsems[p]).wait()

    def issue_scatter(s, p):
        pltpu.async_copy(sbufs[p], out_hbm.at[s, :, wid], ssems[p])

    def wait_scatter(p):
        pltpu.make_async_copy(sbufs[p], out_hbm.at[0, :, 0], ssems[p]).wait()

    basej = [(16 * j + lax.iota(jnp.int32, 16)) * (_BW + 1)
             for j in range(_D // 16)]

    def transpose_add(s, p):
        gb = gbufs[p]
        sb = sbufs[p]
        pv = [pos_v[s, pl.ds(16 * j, 16)] for j in range(_D // 16)]

        @plsc.parallel_loop(0, _BW, 1, unroll=4)
        def _r_body(r):
            rv = jnp.full((16,), r, jnp.int32)
            for j in range(_D // 16):
                v = gb[r, pl.ds(16 * j, 16)]
                plsc.store_scatter(wb, [basej[j] + rv], v + pv[j])

        @plsc.parallel_loop(0, _D, 1, unroll=4)
        def _d_body(d):
            for k in range(_BW // 16):
                sb[d >> 3, d & 7, pl.ds(16 * k, 16)] = (
                    wb[pl.ds(d * (_BW + 1) + 16 * k, 16)])

    def body(s, p):
        @pl.when(s + (_NBUF - 1) < _S)
        def _():
            issue_gather(s + (_NBUF - 1), (p + _NBUF - 1) % _NBUF)

        wait_gather(p)

        @pl.when(s >= _NBUF)
        def _():
            wait_scatter(p)

        transpose_add(s, p)
        issue_scatter(s, p)

    for p in range(_NBUF - 1):
        issue_gather(p, p)

    def outer(i, carry):
        for p in range(_NBUF):
            body(i * _NBUF + p, p)
        return carry

    lax.fori_loop(0, _S // _NBUF, outer, 0)
    for p in range(_NBUF):
        wait_scatter(p)


def kernel(x, emb_table, pos_table):
    xt = jnp.transpose(x.astype(jnp.int32)).reshape(_S, _NW, _BW)
    out5 = _embed_pe(xt, emb_table, pos_table)
    return jnp.transpose(out5, (2, 4, 0, 1, 3)).reshape(_B, _S, _D)
